# Initial kernel scaffold; baseline (speedup 1.0000x reference)
import jax
import jax.numpy as jnp
from jax.experimental import pallas as pl


def kernel(adv_patch, printability_colors):
    raise NotImplementedError("write your pallas kernel here")


# speedup vs baseline: 5.9940x; 5.9940x over previous
import functools

import jax
import jax.numpy as jnp
from jax.experimental import pallas as pl
from jax.experimental.pallas import tpu as pltpu

_K = 512
_ROWS = 392
_BLK = 56


def _quant_kernel(tab_ref, x_ref, o_ref):
    r = x_ref[0]
    g = x_ref[1]
    b = x_ref[2]

    big = jnp.full(r.shape, jnp.inf, jnp.float32)
    zero = jnp.zeros(r.shape, jnp.float32)

    def body(k, carry):
        best, br, bg, bb = carry
        m2r = tab_ref[k, 0]
        m2g = tab_ref[k, 1]
        m2b = tab_ref[k, 2]
        cn = tab_ref[k, 3]
        s = r * m2r + g * m2g + b * m2b + cn
        m = s < best
        best = jnp.where(m, s, best)
        br = jnp.where(m, m2r, br)
        bg = jnp.where(m, m2g, bg)
        bb = jnp.where(m, m2b, bb)
        return best, br, bg, bb

    _, br, bg, bb = jax.lax.fori_loop(
        0, _K, body, (big, zero, zero, zero), unroll=4)

    o_ref[0] = br * -0.5
    o_ref[1] = bg * -0.5
    o_ref[2] = bb * -0.5


@jax.jit
def kernel(adv_patch, printability_colors):
    h, w = adv_patch.shape[-2], adv_patch.shape[-1]
    x = adv_patch.reshape(3, _ROWS, 128)
    c = printability_colors
    cn = jnp.sum(c * c, axis=1, keepdims=True) - 2e-6 * jnp.sum(
        c, axis=1, keepdims=True)
    tab = jnp.concatenate([-2.0 * c, cn], axis=1)

    out = pl.pallas_call(
        _quant_kernel,
        grid=(_ROWS // _BLK,),
        in_specs=[
            pl.BlockSpec(memory_space=pltpu.SMEM),
            pl.BlockSpec((3, _BLK, 128), lambda i: (0, i, 0)),
        ],
        out_specs=pl.BlockSpec((3, _BLK, 128), lambda i: (0, i, 0)),
        out_shape=jax.ShapeDtypeStruct((3, _ROWS, 128), jnp.float32),
    )(tab, x)
    return out.reshape(1, 3, h, w)
